# baseline (speedup 1.0000x reference)
import functools

import jax
import jax.numpy as jnp
from jax import lax
from jax.experimental import pallas as pl
from jax.experimental.pallas import tpu as pltpu
from jax.experimental.pallas import tpu_sc as plsc

D_EMBED = 32
NCHUNK = 512
NBLK_W = NCHUNK // 128


@functools.lru_cache(maxsize=None)
def _make_gather(n: int, s: int):
    info = plsc.get_sparse_core_info()
    nc, ns = info.num_cores, info.num_subcores
    nw = nc * ns
    assert n % (128 * nw) == 0 and n // nw == NCHUNK

    mesh = plsc.VectorSubcoreMesh(core_axis_name="c", subcore_axis_name="s")

    @functools.partial(
        pl.kernel,
        mesh=mesh,
        out_type=jax.ShapeDtypeStruct((s * 4, n // 128, 8, 128), jnp.float32),
        scratch_types=(
            [pltpu.VMEM((NBLK_W, 128), jnp.int32) for _ in range(2)]
            + [pltpu.VMEM((NBLK_W, 128, D_EMBED), jnp.float32) for _ in range(2)]
            + [pltpu.SemaphoreType.DMA for _ in range(6)]
        ),
        compiler_params=pltpu.CompilerParams(
            use_tc_tiling_on_sc=False, needs_layout_passes=False),
    )
    def gather_kernel(idx_hbm, table_hbm, out_hbm, *scr):
        idx_v = scr[0:2]
        rows_v = scr[2:4]
        sem_idx = scr[4:6]
        sem_g = scr[6:8]
        sem_st = scr[8:10]

        wid = lax.axis_index("s") * nc + lax.axis_index("c")

        def idx_slice(col):
            return idx_hbm.at[pl.ds(col * (n // 128) + wid * NBLK_W, NBLK_W), :]

        def wait_idx(b):
            pltpu.make_async_copy(idx_slice(0), idx_v[b], sem_idx[b]).wait()

        def wait_gathers(b):
            for q in range(NBLK_W):
                pltpu.make_async_copy(
                    table_hbm.at[pl.ds(0, 128)], rows_v[b].at[q], sem_g[b]
                ).wait()

        def wait_stores(b):
            for q in range(NBLK_W):
                pltpu.make_async_copy(
                    rows_v[b].at[q],
                    out_hbm.at[0, pl.ds(0, 128), 0, pl.ds(0, D_EMBED)],
                    sem_st[b],
                ).wait()

        def start_gathers(b):
            for q in range(NBLK_W):
                pltpu.async_copy(
                    table_hbm.at[idx_v[b].at[q]], rows_v[b].at[q], sem_g[b])

        def start_stores(col, b):
            for c in range(D_EMBED):
                g, cs = divmod(c, 8)
                pltpu.async_copy(
                    rows_v[b].at[:, :, c],
                    out_hbm.at[col * 4 + g, pl.ds(wid * NBLK_W, NBLK_W), cs, :],
                    sem_st[b],
                )

        for b in range(2):
            pltpu.async_copy(idx_slice(b), idx_v[b], sem_idx[b])

        def group(grp, carry):
            for b in range(2):
                i = grp * 2 + b
                p = 1 - b
                wait_idx(b)

                @pl.when(grp > 0)
                def _():
                    wait_stores(b)

                start_gathers(b)

                @pl.when(i > 0)
                def _():
                    wait_gathers(p)

                    @pl.when(i + 1 < s)
                    def _():
                        pltpu.async_copy(idx_slice(i + 1), idx_v[p], sem_idx[p])

                    start_stores(i - 1, p)
            return carry

        lax.fori_loop(0, s // 2, group, 0)

        wait_gathers(1)
        start_stores(s - 1, 1)
        for b in range(2):
            wait_stores(b)

    return gather_kernel


def kernel(sensor_id, table):
    n, s = sensor_id.shape
    idx2 = sensor_id.T.reshape(s * (n // 128), 128)
    o4 = _make_gather(n, s)(idx2, table)
    o5 = o4.reshape(s, 4, n // 128, 8, 128)
    return o5.transpose(2, 4, 0, 1, 3).reshape(n, s, D_EMBED)


# speedup vs baseline: 92.9118x; 92.9118x over previous
import functools

import jax
import jax.numpy as jnp
from jax import lax
from jax.experimental import pallas as pl
from jax.experimental.pallas import tpu as pltpu
from jax.experimental.pallas import tpu_sc as plsc

D_EMBED = 32
NCHUNK = 512
NBLK_W = NCHUNK // 128


@functools.lru_cache(maxsize=None)
def _make_gather(n: int, s: int):
    info = plsc.get_sparse_core_info()
    nc, ns = info.num_cores, info.num_subcores
    nw = nc * ns
    assert n % (128 * nw) == 0 and n // nw == NCHUNK

    mesh = plsc.VectorSubcoreMesh(core_axis_name="c", subcore_axis_name="s")

    @functools.partial(
        pl.kernel,
        mesh=mesh,
        out_type=jax.ShapeDtypeStruct((s * (n // 128), 128, D_EMBED),
                                      jnp.float32),
        scratch_types=(
            [pltpu.VMEM((NBLK_W, 128), jnp.int32) for _ in range(2)]
            + [pltpu.VMEM((NBLK_W, 128, D_EMBED), jnp.float32) for _ in range(2)]
            + [pltpu.SemaphoreType.DMA for _ in range(6)]
        ),
        compiler_params=pltpu.CompilerParams(
            use_tc_tiling_on_sc=False, needs_layout_passes=False),
    )
    def gather_kernel(idx_hbm, table_hbm, out_hbm, *scr):
        idx_v = scr[0:2]
        rows_v = scr[2:4]
        sem_idx = scr[4:6]
        sem_g = scr[6:8]
        sem_st = scr[8:10]

        wid = lax.axis_index("s") * nc + lax.axis_index("c")

        def idx_slice(col):
            return idx_hbm.at[pl.ds(col * (n // 128) + wid * NBLK_W, NBLK_W), :]

        def wait_idx(b):
            pltpu.make_async_copy(idx_slice(0), idx_v[b], sem_idx[b]).wait()

        def wait_gathers(b):
            for q in range(NBLK_W):
                pltpu.make_async_copy(
                    table_hbm.at[pl.ds(0, 128)], rows_v[b].at[q], sem_g[b]
                ).wait()

        def wait_stores(b):
            for q in range(NBLK_W):
                pltpu.make_async_copy(
                    rows_v[b].at[q], out_hbm.at[0], sem_st[b]
                ).wait()

        def start_gathers(b):
            for q in range(NBLK_W):
                pltpu.async_copy(
                    table_hbm.at[idx_v[b].at[q]], rows_v[b].at[q], sem_g[b])

        def start_stores(col, b):
            for q in range(NBLK_W):
                pltpu.async_copy(
                    rows_v[b].at[q],
                    out_hbm.at[col * (n // 128) + wid * NBLK_W + q],
                    sem_st[b],
                )

        for b in range(2):
            pltpu.async_copy(idx_slice(b), idx_v[b], sem_idx[b])

        def group(grp, carry):
            for b in range(2):
                i = grp * 2 + b
                p = 1 - b
                wait_idx(b)

                @pl.when(grp > 0)
                def _():
                    wait_stores(b)

                start_gathers(b)

                @pl.when(i > 0)
                def _():
                    wait_gathers(p)

                    @pl.when(i + 1 < s)
                    def _():
                        pltpu.async_copy(idx_slice(i + 1), idx_v[p], sem_idx[p])

                    start_stores(i - 1, p)
            return carry

        lax.fori_loop(0, s // 2, group, 0)

        wait_gathers(1)
        start_stores(s - 1, 1)
        for b in range(2):
            wait_stores(b)

    return gather_kernel


def _tc_transform(x_ref, o_ref):
    x = x_ref[0]
    b = jnp.swapaxes(x, 1, 2)
    o_ref[...] = b.reshape(128, 4, 8, 128).transpose(1, 0, 2, 3)


@functools.lru_cache(maxsize=None)
def _make_transform(n: int, s: int):
    return pl.pallas_call(
        _tc_transform,
        grid=(s,),
        in_specs=[
            pl.BlockSpec((1, n // 128, 128, D_EMBED),
                         lambda i: (i, 0, 0, 0)),
        ],
        out_specs=pl.BlockSpec((4, n // 128, 8, 128),
                               lambda i: (i, 0, 0, 0)),
        out_shape=jax.ShapeDtypeStruct((s * 4, n // 128, 8, 128),
                                       jnp.float32),
    )


def kernel(sensor_id, table):
    n, s = sensor_id.shape
    idx2 = sensor_id.T.reshape(s * (n // 128), 128)
    inter = _make_gather(n, s)(idx2, table)
    o4 = _make_transform(n, s)(inter.reshape(s, n // 128, 128, D_EMBED))
    o5 = o4.reshape(s, 4, n // 128, 8, 128)
    return o5.transpose(2, 4, 0, 1, 3).reshape(n, s, D_EMBED)
